# baseline (speedup 1.0000x reference)
import functools

import jax
import jax.numpy as jnp
from jax import lax
from jax.experimental import pallas as pl
from jax.experimental.pallas import tpu as pltpu
from jax.experimental.pallas import tpu_sc as plsc

B = 2
N = 1000
C = 81
MH = 14
MW = 14
MROW = MH * MW * C
MAX_OUT = 100
NMS_THR = 0.3
MIN_CONF = 0.5
STD = (0.1, 0.1, 0.2, 0.2)


def _tc_body(win_ref, probs_ref, deltas_ref, rois_ref, det_ref, idxc_ref, sclc_ref):
    wy1 = win_ref[0]
    wx1 = win_ref[1]
    wy2 = win_ref[2]
    wx2 = win_ref[3]

    best = probs_ref[0]
    cid = jnp.zeros((B, N), jnp.int32)
    d0 = deltas_ref[:, 0, 0, :]
    d1 = deltas_ref[:, 0, 1, :]
    d2 = deltas_ref[:, 0, 2, :]
    d3 = deltas_ref[:, 0, 3, :]
    for c in range(1, C):
        pc = probs_ref[c]
        upd = pc > best
        best = jnp.where(upd, pc, best)
        cid = jnp.where(upd, c, cid)
        d0 = jnp.where(upd, deltas_ref[:, c, 0, :], d0)
        d1 = jnp.where(upd, deltas_ref[:, c, 1, :], d1)
        d2 = jnp.where(upd, deltas_ref[:, c, 2, :], d2)
        d3 = jnp.where(upd, deltas_ref[:, c, 3, :], d3)

    ry1 = rois_ref[:, 0, :]
    rx1 = rois_ref[:, 1, :]
    ry2 = rois_ref[:, 2, :]
    rx2 = rois_ref[:, 3, :]
    nz = (jnp.abs(ry1) + jnp.abs(rx1) + jnp.abs(ry2) + jnp.abs(rx2)) != 0.0

    height = ry2 - ry1
    width = rx2 - rx1
    cy = ry1 + 0.5 * height + (d0 * STD[0]) * height
    cx = rx1 + 0.5 * width + (d1 * STD[1]) * width
    height = height * jnp.exp(d2 * STD[2])
    width = width * jnp.exp(d3 * STD[3])
    y1 = cy - 0.5 * height
    x1 = cx - 0.5 * width
    y2 = y1 + height
    x2 = x1 + width
    y1 = jnp.clip(y1, wy1, wy2)
    x1 = jnp.clip(x1, wx1, wx2)
    y2 = jnp.clip(y2, wy1, wy2)
    x2 = jnp.clip(x2, wx1, wx2)
    area = (y2 - y1) * (x2 - x1)

    valid = nz & (cid > 0) & (best >= MIN_CONF)
    key0 = jnp.where(valid, best, -1.0)
    cidf = cid.astype(jnp.float32)

    lin = lax.broadcasted_iota(jnp.int32, (B, N), 1)
    lane = lax.broadcasted_iota(jnp.int32, (B, 1, 128), 2)
    fields = jnp.stack([y1, x1, y2, x2, cidf, best, area], axis=1)

    def step(t, carry):
        key, top, cnt = carry
        m = jnp.max(key, axis=1, keepdims=True)
        alive = m >= 0.0
        idx = jnp.min(jnp.where(key == m, lin, N), axis=1, keepdims=True)
        sel = lin == idx
        g = jnp.sum(jnp.where(sel[:, None, :], fields, 0.0), axis=2)
        gy1 = g[:, 0:1]
        gx1 = g[:, 1:2]
        gy2 = g[:, 2:3]
        gx2 = g[:, 3:4]
        gcf = g[:, 4:5]
        gs = g[:, 5:6]
        ga = g[:, 6:7]
        yy1 = jnp.maximum(gy1, y1)
        xx1 = jnp.maximum(gx1, x1)
        yy2 = jnp.minimum(gy2, y2)
        xx2 = jnp.minimum(gx2, x2)
        inter = jnp.maximum(0.0, yy2 - yy1) * jnp.maximum(0.0, xx2 - xx1)
        iou = inter / (ga + area - inter + 1e-12)
        supp = (cidf == gcf) & (iou > NMS_THR)
        key = jnp.where(alive, jnp.where(supp | sel, -1.0, key), key)
        top = jnp.where(alive & (lin == t), idx, top)
        cnt = cnt + jnp.where(alive, 1, 0)
        row = jnp.where(
            lane == 0, gy1[:, :, None],
            jnp.where(lane == 1, gx1[:, :, None],
                      jnp.where(lane == 2, gy2[:, :, None],
                                jnp.where(lane == 3, gx2[:, :, None],
                                          jnp.where(lane == 4, gcf[:, :, None],
                                                    jnp.where(lane == 5,
                                                              gs[:, :, None], 0.0))))))
        row = jnp.where(alive[:, :, None], row, 0.0)
        det_ref[:, pl.ds(t, 1), :] = row
        return key, top, cnt

    top0 = jnp.zeros((B, N), jnp.int32)
    cnt0 = jnp.zeros((B, 1), jnp.int32)
    _, top, cnt = lax.fori_loop(0, MAX_OUT, step, (key0, top0, cnt0), unroll=2)
    det_ref[:, pl.ds(MAX_OUT, 1), :] = (
        cnt.astype(jnp.float32)[:, :, None] + jnp.zeros((B, 1, 128), jnp.float32)
    )
    scl = jnp.where(lane[:, 0, :] < cnt, 1.0, 0.0)
    for b in range(B):
        for j, o in enumerate(_OFFS):
            idxc_ref[pl.ds(b * _NCH + j, 1), :] = top[b:b + 1, o:o + 16]
            sclc_ref[pl.ds(b * _NCH + j, 1), :] = scl[b:b + 1, o:o + 16]


def _tc_call(probs_t, deltas_t, rois_t, window):
    return pl.pallas_call(
        _tc_body,
        in_specs=[
            pl.BlockSpec(memory_space=pltpu.SMEM),
            pl.BlockSpec((C, B, N), lambda: (0, 0, 0)),
            pl.BlockSpec((B, C, 4, N), lambda: (0, 0, 0, 0)),
            pl.BlockSpec((B, 4, N), lambda: (0, 0, 0)),
        ],
        out_specs=[
            pl.BlockSpec((B, MAX_OUT + 1, 128), lambda: (0, 0, 0)),
            pl.BlockSpec((B * _NCH, 16), lambda: (0, 0)),
            pl.BlockSpec((B * _NCH, 16), lambda: (0, 0)),
        ],
        out_shape=[
            jax.ShapeDtypeStruct((B, MAX_OUT + 1, 128), jnp.float32),
            jax.ShapeDtypeStruct((B * _NCH, 16), jnp.int32),
            jax.ShapeDtypeStruct((B * _NCH, 16), jnp.float32),
        ],
    )(window, probs_t, deltas_t, rois_t)


_OFFS = (0, 16, 32, 48, 64, 80, 96)
_NCH = len(_OFFS)
_OW = 112
_BLK = 8
_NBLK = MROW // _BLK
_BPT = _NBLK // 32


_NPH = 4
assert _BPT >= 2 * _NPH


def _sc_body(src, idx_hbm, scl_hbm, out, idxv, sclv, *bufs):
    wid = lax.axis_index("s") * 2 + lax.axis_index("c")
    nb = _NPH * B
    ibs = tuple(tuple(bufs[0 * nb + p * B + b] for b in range(B)) for p in range(_NPH))
    obs = tuple(tuple(bufs[1 * nb + p * B + b] for b in range(B)) for p in range(_NPH))
    sis = tuple(tuple(bufs[2 * nb + p * B + b] for b in range(B)) for p in range(_NPH))
    sos = tuple(tuple(bufs[3 * nb + p * B + b] for b in range(B)) for p in range(_NPH))
    pltpu.sync_copy(idx_hbm, idxv)
    pltpu.sync_copy(scl_hbm, sclv)
    zero = jnp.zeros((16,), jnp.int32)

    def process(ph, nrows):
        for b in range(B):
            ib = ibs[ph][b]
            ob = obs[ph][b]
            for r in range(nrows):
                rv = zero + r
                for j in range(_NCH):
                    g = plsc.load_gather(ib, [rv, idxv[b * _NCH + j]])
                    ob[r, pl.ds(_OFFS[j], 16)] = g * sclv[b * _NCH + j]

    base0 = wid * _BPT * _BLK
    for p in range(_NPH):
        for b in range(B):
            pltpu.make_async_copy(
                src.at[pl.ds(base0 + p * _BLK, _BLK), b], ibs[p][b],
                sis[p][b]).start()

    def outer(i, _):
        for ph in range(_NPH):
            j = _NPH * i + ph
            rbase = base0 + j * _BLK
            for b in range(B):
                pltpu.make_async_copy(
                    src.at[pl.ds(rbase, _BLK), b], ibs[ph][b], sis[ph][b]).wait()

            @pl.when(j >= _NPH)
            def _():
                for b in range(B):
                    pltpu.make_async_copy(
                        obs[ph][b], out.at[pl.ds(rbase - _NPH * _BLK, _BLK), b],
                        sos[ph][b]).wait()

            process(ph, _BLK)
            for b in range(B):
                pltpu.make_async_copy(
                    obs[ph][b], out.at[pl.ds(rbase, _BLK), b], sos[ph][b]).start()

            @pl.when(j + _NPH < _BPT)
            def _():
                for b in range(B):
                    pltpu.make_async_copy(
                        src.at[pl.ds(rbase + _NPH * _BLK, _BLK), b],
                        ibs[ph][b], sis[ph][b]).start()

        return 0

    lax.fori_loop(0, _BPT // _NPH, outer, 0)
    for j in range(_BPT - _BPT % _NPH, _BPT):
        ph = j % _NPH
        rbase = base0 + j * _BLK
        for b in range(B):
            pltpu.make_async_copy(
                src.at[pl.ds(rbase, _BLK), b], ibs[ph][b], sis[ph][b]).wait()
        for b in range(B):
            pltpu.make_async_copy(
                obs[ph][b], out.at[pl.ds(rbase - _NPH * _BLK, _BLK), b],
                sos[ph][b]).wait()
        process(ph, _BLK)
        for b in range(B):
            pltpu.make_async_copy(
                obs[ph][b], out.at[pl.ds(rbase, _BLK), b], sos[ph][b]).start()
    for j in range(_BPT - _NPH, _BPT):
        ph = j % _NPH
        rbase = base0 + j * _BLK
        for b in range(B):
            pltpu.make_async_copy(
                obs[ph][b], out.at[pl.ds(rbase, _BLK), b], sos[ph][b]).wait()

    @pl.when(wid == 0)
    def _():
        tbase = 32 * _BPT * _BLK
        for b in range(B):
            pltpu.sync_copy(src.at[pl.ds(tbase, 4), b], ibs[0][b].at[pl.ds(0, 4)])
        process(0, 4)
        for b in range(B):
            pltpu.sync_copy(obs[0][b].at[pl.ds(0, 4)], out.at[pl.ds(tbase, 4), b])


@functools.lru_cache(maxsize=1)
def _sc_gather_fn():
    return pl.kernel(
        _sc_body,
        out_type=jax.ShapeDtypeStruct((MROW, B, _OW), jnp.float32),
        mesh=plsc.VectorSubcoreMesh(core_axis_name="c", subcore_axis_name="s"),
        compiler_params=pltpu.CompilerParams(needs_layout_passes=False),
        scratch_types=(
            [pltpu.VMEM((B * _NCH, 16), jnp.int32),
             pltpu.VMEM((B * _NCH, 16), jnp.float32)]
            + [pltpu.VMEM((_BLK, N), jnp.float32)] * (_NPH * B)
            + [pltpu.VMEM((_BLK, _OW), jnp.float32)] * (_NPH * B)
            + [pltpu.SemaphoreType.DMA] * (2 * _NPH * B)
        ),
    )


@jax.jit
def kernel(rois, probs, deltas, masks, window):
    probs_t = probs.transpose(2, 0, 1)
    deltas_t = deltas.transpose(0, 2, 3, 1)
    rois_t = rois.transpose(0, 2, 1)

    det_out, idx_chunks, scl_chunks = _tc_call(probs_t, deltas_t, rois_t, window)

    dets = det_out[:, :MAX_OUT, :6]
    src = masks.transpose(2, 3, 4, 0, 1).reshape(MROW, B, N)
    out3 = _sc_gather_fn()(src, idx_chunks, scl_chunks)
    mk = (
        out3[:, :, :MAX_OUT]
        .reshape(MH, MW, C, B, MAX_OUT)
        .transpose(3, 4, 0, 1, 2)
    )
    return dets, mk


# speedup vs baseline: 1.0308x; 1.0308x over previous
import functools

import jax
import jax.numpy as jnp
from jax import lax
from jax.experimental import pallas as pl
from jax.experimental.pallas import tpu as pltpu
from jax.experimental.pallas import tpu_sc as plsc

B = 2
N = 1000
C = 81
MH = 14
MW = 14
MROW = MH * MW * C
MAX_OUT = 100
NMS_THR = 0.3
MIN_CONF = 0.5
STD = (0.1, 0.1, 0.2, 0.2)


def _tc_body(win_ref, probs_ref, deltas_ref, rois_ref, det_ref, idxc_ref, sclc_ref):
    wy1 = win_ref[0]
    wx1 = win_ref[1]
    wy2 = win_ref[2]
    wx2 = win_ref[3]

    best = probs_ref[0]
    cid = jnp.zeros((B, N), jnp.int32)
    d0 = deltas_ref[:, 0, 0, :]
    d1 = deltas_ref[:, 0, 1, :]
    d2 = deltas_ref[:, 0, 2, :]
    d3 = deltas_ref[:, 0, 3, :]
    for c in range(1, C):
        pc = probs_ref[c]
        upd = pc > best
        best = jnp.where(upd, pc, best)
        cid = jnp.where(upd, c, cid)
        d0 = jnp.where(upd, deltas_ref[:, c, 0, :], d0)
        d1 = jnp.where(upd, deltas_ref[:, c, 1, :], d1)
        d2 = jnp.where(upd, deltas_ref[:, c, 2, :], d2)
        d3 = jnp.where(upd, deltas_ref[:, c, 3, :], d3)

    ry1 = rois_ref[:, 0, :]
    rx1 = rois_ref[:, 1, :]
    ry2 = rois_ref[:, 2, :]
    rx2 = rois_ref[:, 3, :]
    nz = (jnp.abs(ry1) + jnp.abs(rx1) + jnp.abs(ry2) + jnp.abs(rx2)) != 0.0

    height = ry2 - ry1
    width = rx2 - rx1
    cy = ry1 + 0.5 * height + (d0 * STD[0]) * height
    cx = rx1 + 0.5 * width + (d1 * STD[1]) * width
    height = height * jnp.exp(d2 * STD[2])
    width = width * jnp.exp(d3 * STD[3])
    y1 = cy - 0.5 * height
    x1 = cx - 0.5 * width
    y2 = y1 + height
    x2 = x1 + width
    y1 = jnp.clip(y1, wy1, wy2)
    x1 = jnp.clip(x1, wx1, wx2)
    y2 = jnp.clip(y2, wy1, wy2)
    x2 = jnp.clip(x2, wx1, wx2)
    area = (y2 - y1) * (x2 - x1)

    valid = nz & (cid > 0) & (best >= MIN_CONF)
    key0 = jnp.where(valid, best, -1.0)
    cidf = cid.astype(jnp.float32)

    lin = lax.broadcasted_iota(jnp.int32, (B, N), 1)
    lane = lax.broadcasted_iota(jnp.int32, (B, 1, 128), 2)
    fields = jnp.stack([y1, x1, y2, x2, cidf, best, area], axis=1)

    def step(t, carry):
        key, top, cnt = carry
        m = jnp.max(key, axis=1, keepdims=True)
        alive = m >= 0.0
        idx = jnp.min(jnp.where(key == m, lin, N), axis=1, keepdims=True)
        sel = lin == idx
        g = jnp.sum(jnp.where(sel[:, None, :], fields, 0.0), axis=2)
        gy1 = g[:, 0:1]
        gx1 = g[:, 1:2]
        gy2 = g[:, 2:3]
        gx2 = g[:, 3:4]
        gcf = g[:, 4:5]
        gs = g[:, 5:6]
        ga = g[:, 6:7]
        yy1 = jnp.maximum(gy1, y1)
        xx1 = jnp.maximum(gx1, x1)
        yy2 = jnp.minimum(gy2, y2)
        xx2 = jnp.minimum(gx2, x2)
        inter = jnp.maximum(0.0, yy2 - yy1) * jnp.maximum(0.0, xx2 - xx1)
        iou = inter / (ga + area - inter + 1e-12)
        supp = (cidf == gcf) & (iou > NMS_THR)
        key = jnp.where(alive, jnp.where(supp | sel, -1.0, key), key)
        top = jnp.where(alive & (lin == t), idx, top)
        cnt = cnt + jnp.where(alive, 1, 0)
        row = jnp.where(
            lane == 0, gy1[:, :, None],
            jnp.where(lane == 1, gx1[:, :, None],
                      jnp.where(lane == 2, gy2[:, :, None],
                                jnp.where(lane == 3, gx2[:, :, None],
                                          jnp.where(lane == 4, gcf[:, :, None],
                                                    jnp.where(lane == 5,
                                                              gs[:, :, None], 0.0))))))
        row = jnp.where(alive[:, :, None], row, 0.0)
        det_ref[:, pl.ds(t, 1), :] = row
        return key, top, cnt

    top0 = jnp.zeros((B, N), jnp.int32)
    cnt0 = jnp.zeros((B, 1), jnp.int32)
    _, top, cnt = lax.fori_loop(0, MAX_OUT, step, (key0, top0, cnt0), unroll=2)
    det_ref[:, pl.ds(MAX_OUT, 1), :] = (
        cnt.astype(jnp.float32)[:, :, None] + jnp.zeros((B, 1, 128), jnp.float32)
    )
    scl = jnp.where(lane[:, 0, :] < cnt, 1.0, 0.0)
    for b in range(B):
        for j, o in enumerate(_OFFS):
            idxc_ref[pl.ds(b * _NCH + j, 1), :] = top[b:b + 1, o:o + 16]
            sclc_ref[pl.ds(b * _NCH + j, 1), :] = scl[b:b + 1, o:o + 16]


def _tc_call(probs_t, deltas_t, rois_t, window):
    return pl.pallas_call(
        _tc_body,
        in_specs=[
            pl.BlockSpec(memory_space=pltpu.SMEM),
            pl.BlockSpec((C, B, N), lambda: (0, 0, 0)),
            pl.BlockSpec((B, C, 4, N), lambda: (0, 0, 0, 0)),
            pl.BlockSpec((B, 4, N), lambda: (0, 0, 0)),
        ],
        out_specs=[
            pl.BlockSpec((B, MAX_OUT + 1, 128), lambda: (0, 0, 0)),
            pl.BlockSpec((B * _NCH, 16), lambda: (0, 0)),
            pl.BlockSpec((B * _NCH, 16), lambda: (0, 0)),
        ],
        out_shape=[
            jax.ShapeDtypeStruct((B, MAX_OUT + 1, 128), jnp.float32),
            jax.ShapeDtypeStruct((B * _NCH, 16), jnp.int32),
            jax.ShapeDtypeStruct((B * _NCH, 16), jnp.float32),
        ],
    )(window, probs_t, deltas_t, rois_t)


_OFFS = (0, 16, 32, 48, 64, 80, 96)
_NCH = len(_OFFS)
_OW = 112
_BLK = 8
_NBLK = MROW // _BLK
_BPT = _NBLK // 32


_NPH = 2
assert _BPT >= 2 * _NPH


def _sc_body(src, idx_hbm, scl_hbm, out, idxv, sclv, *bufs):
    wid = lax.axis_index("s") * 2 + lax.axis_index("c")
    nb = _NPH * B
    ibs = tuple(tuple(bufs[0 * nb + p * B + b] for b in range(B)) for p in range(_NPH))
    obs = tuple(tuple(bufs[1 * nb + p * B + b] for b in range(B)) for p in range(_NPH))
    sis = tuple(tuple(bufs[2 * nb + p * B + b] for b in range(B)) for p in range(_NPH))
    sos = tuple(tuple(bufs[3 * nb + p * B + b] for b in range(B)) for p in range(_NPH))
    pltpu.sync_copy(idx_hbm, idxv)
    pltpu.sync_copy(scl_hbm, sclv)
    zero = jnp.zeros((16,), jnp.int32)

    def process(ph, nrows):
        for b in range(B):
            ib = ibs[ph][b]
            ob = obs[ph][b]
            for r in range(nrows):
                rv = zero + r
                for j in range(_NCH):
                    g = plsc.load_gather(ib, [rv, idxv[b * _NCH + j]])
                    ob[r, pl.ds(_OFFS[j], 16)] = g * sclv[b * _NCH + j]

    base0 = wid * _BPT * _BLK
    for p in range(_NPH):
        for b in range(B):
            pltpu.make_async_copy(
                src.at[pl.ds(base0 + p * _BLK, _BLK), b], ibs[p][b],
                sis[p][b]).start()

    def outer(i, _):
        for ph in range(_NPH):
            j = _NPH * i + ph
            rbase = base0 + j * _BLK
            for b in range(B):
                pltpu.make_async_copy(
                    src.at[pl.ds(rbase, _BLK), b], ibs[ph][b], sis[ph][b]).wait()

            @pl.when(j >= _NPH)
            def _():
                for b in range(B):
                    pltpu.make_async_copy(
                        obs[ph][b], out.at[pl.ds(rbase - _NPH * _BLK, _BLK), b],
                        sos[ph][b]).wait()

            process(ph, _BLK)
            for b in range(B):
                pltpu.make_async_copy(
                    obs[ph][b], out.at[pl.ds(rbase, _BLK), b], sos[ph][b]).start()

            @pl.when(j + _NPH < _BPT)
            def _():
                for b in range(B):
                    pltpu.make_async_copy(
                        src.at[pl.ds(rbase + _NPH * _BLK, _BLK), b],
                        ibs[ph][b], sis[ph][b]).start()

        return 0

    lax.fori_loop(0, _BPT // _NPH, outer, 0)
    for j in range(_BPT - _BPT % _NPH, _BPT):
        ph = j % _NPH
        rbase = base0 + j * _BLK
        for b in range(B):
            pltpu.make_async_copy(
                src.at[pl.ds(rbase, _BLK), b], ibs[ph][b], sis[ph][b]).wait()
        for b in range(B):
            pltpu.make_async_copy(
                obs[ph][b], out.at[pl.ds(rbase - _NPH * _BLK, _BLK), b],
                sos[ph][b]).wait()
        process(ph, _BLK)
        for b in range(B):
            pltpu.make_async_copy(
                obs[ph][b], out.at[pl.ds(rbase, _BLK), b], sos[ph][b]).start()
    for j in range(_BPT - _NPH, _BPT):
        ph = j % _NPH
        rbase = base0 + j * _BLK
        for b in range(B):
            pltpu.make_async_copy(
                obs[ph][b], out.at[pl.ds(rbase, _BLK), b], sos[ph][b]).wait()

    @pl.when(wid == 0)
    def _():
        tbase = 32 * _BPT * _BLK
        for b in range(B):
            pltpu.sync_copy(src.at[pl.ds(tbase, 4), b], ibs[0][b].at[pl.ds(0, 4)])
        process(0, 4)
        for b in range(B):
            pltpu.sync_copy(obs[0][b].at[pl.ds(0, 4)], out.at[pl.ds(tbase, 4), b])


@functools.lru_cache(maxsize=1)
def _sc_gather_fn():
    return pl.kernel(
        _sc_body,
        out_type=jax.ShapeDtypeStruct((MROW, B, _OW), jnp.float32),
        mesh=plsc.VectorSubcoreMesh(core_axis_name="c", subcore_axis_name="s"),
        compiler_params=pltpu.CompilerParams(needs_layout_passes=False),
        scratch_types=(
            [pltpu.VMEM((B * _NCH, 16), jnp.int32),
             pltpu.VMEM((B * _NCH, 16), jnp.float32)]
            + [pltpu.VMEM((_BLK, N), jnp.float32)] * (_NPH * B)
            + [pltpu.VMEM((_BLK, _OW), jnp.float32)] * (_NPH * B)
            + [pltpu.SemaphoreType.DMA] * (2 * _NPH * B)
        ),
    )


@jax.jit
def kernel(rois, probs, deltas, masks, window):
    probs_t = probs.transpose(2, 0, 1)
    deltas_t = deltas.transpose(0, 2, 3, 1)
    rois_t = rois.transpose(0, 2, 1)

    det_out, idx_chunks, scl_chunks = _tc_call(probs_t, deltas_t, rois_t, window)

    dets = det_out[:, :MAX_OUT, :6]
    src = masks.transpose(2, 3, 4, 0, 1).reshape(MROW, B, N)
    out3 = _sc_gather_fn()(src, idx_chunks, scl_chunks)
    mk = (
        out3[:, :, :MAX_OUT]
        .reshape(MH, MW, C, B, MAX_OUT)
        .transpose(3, 4, 0, 1, 2)
    )
    return dets, mk
